# baseline (speedup 1.0000x reference)
import functools
import math

import jax
import jax.numpy as jnp
from jax import lax
from jax.experimental import pallas as pl
from jax.experimental.pallas import tpu as pltpu
from jax.experimental.pallas import tpu_sc as plsc

_N = 2048
_SIZE = 32000
_CONF = 0.9
_FILL = 0.1 / (_SIZE - 2)
_C = _CONF * math.log(_CONF) + 0.1 * math.log(_FILL)


_BR = 256
_BC = 6400
_GR = _N // _BR
_GC = _SIZE // _BC


def _tc_body(tgt_ref, x_ref, out_ref):
    i = pl.program_id(0)
    j = pl.program_id(1)

    tgt = tgt_ref[0]
    rowcoef = jnp.where(tgt == 0, 0.0, -_FILL).astype(jnp.float32)
    prod = lax.dot_general(
        rowcoef, x_ref[...],
        dimension_numbers=(((0,), (0,)), ((), ())),
        preferred_element_type=jnp.float32,
    )

    @pl.when((i == 0) & (j == 0))
    def _init():
        out_ref[0, 0] = 0.0

    out_ref[0, 0] += jnp.sum(prod)


def _tc_call(tgt3, x):
    return pl.pallas_call(
        _tc_body,
        grid=(_GR, _GC),
        in_specs=[
            pl.BlockSpec((1, _BR, 1), lambda i, j: (i, 0, 0)),
            pl.BlockSpec((_BR, _BC), lambda i, j: (i, j)),
        ],
        out_specs=pl.BlockSpec(
            (1, 1), lambda i, j: (0, 0), memory_space=pltpu.SMEM
        ),
        out_shape=jax.ShapeDtypeStruct((1, 1), jnp.float32),
    )(tgt3, x)


_NC = 2
_NS = 16
_L = 16
_NW = _NC * _NS
_RPW = _N // _NW

_sc_mesh = plsc.VectorSubcoreMesh(core_axis_name="c", subcore_axis_name="s")


@functools.partial(
    pl.kernel,
    mesh=_sc_mesh,
    out_type=jax.ShapeDtypeStruct((_NW * _L,), jnp.float32),
    scratch_types=[
        pltpu.VMEM((_RPW,), jnp.int32),
        pltpu.VMEM((_RPW,), jnp.int32),
        pltpu.VMEM((_RPW,), jnp.int32),
        pltpu.VMEM((_RPW,), jnp.float32),
        pltpu.VMEM((_RPW,), jnp.float32),
        pltpu.VMEM((_L,), jnp.float32),
        pltpu.SemaphoreType.DMA,
    ],
)
def _sc_kernel(xf_hbm, tgt_hbm, out_hbm, tgt_v, idxt_v, idx0_v,
               valt_v, val0_v, acc_v, sem):
    wid = lax.axis_index("s") * _NC + lax.axis_index("c")
    base = wid * _RPW
    pltpu.sync_copy(tgt_hbm.at[pl.ds(base, _RPW)], tgt_v)

    for k in range(_RPW // _L):
        t = tgt_v[pl.ds(k * _L, _L)]
        i_vec = lax.iota(jnp.int32, _L) + (base + k * _L)
        flat = i_vec * _SIZE
        idxt_v[pl.ds(k * _L, _L)] = flat + t
        idx0_v[pl.ds(k * _L, _L)] = flat

    pltpu.async_copy(xf_hbm.at[idxt_v], valt_v, sem).wait()
    pltpu.async_copy(xf_hbm.at[idx0_v], val0_v, sem).wait()

    acc = jnp.zeros((_L,), jnp.float32)
    for k in range(_RPW // _L):
        t = tgt_v[pl.ds(k * _L, _L)]
        vt = valt_v[pl.ds(k * _L, _L)]
        v0 = val0_v[pl.ds(k * _L, _L)]
        contrib = _C - (_CONF - _FILL) * vt + _FILL * v0
        acc = acc + jnp.where(t != 0, contrib, 0.0)

    acc_v[...] = acc
    pltpu.sync_copy(acc_v, out_hbm.at[pl.ds(wid * _L, _L)])


def kernel(x, target):
    tgt3 = target.reshape(_GR, _BR, 1)
    dense = _tc_call(tgt3, x)
    sparse = _sc_kernel(x.reshape(-1), target)
    return dense[0, 0] + jnp.sum(sparse)


# speedup vs baseline: 2.9438x; 2.9438x over previous
import math

import jax
import jax.numpy as jnp
from jax import lax
from jax.experimental import pallas as pl
from jax.experimental.pallas import tpu as pltpu

_N = 2048
_SIZE = 32000
_CONF = 0.9
_FILL = 0.1 / (_SIZE - 2)
_C = _CONF * math.log(_CONF) + 0.1 * math.log(_FILL)

_BR = 128
_GR = _N // _BR


def _body(tgt_ref, x_ref, out_ref):
    i = pl.program_id(0)

    x = x_ref[...]
    tgt = tgt_ref[0]
    live = tgt != 0
    rowcoef = jnp.where(live, -_FILL, 0.0).astype(jnp.float32)

    dense = lax.dot_general(
        rowcoef, x,
        dimension_numbers=(((0,), (0,)), ((), ())),
        preferred_element_type=jnp.float32,
    )

    teff = jnp.where(live, tgt, -1)
    col = lax.broadcasted_iota(jnp.int32, (_BR, _SIZE), 1)
    s_t = jnp.sum(jnp.where(col == teff, x, 0.0))

    c0 = jnp.sum(jnp.where(live, x_ref[:, 0:1], 0.0))
    cnt = jnp.sum(live.astype(jnp.float32))

    partial = (jnp.sum(dense) - (_CONF - _FILL) * s_t
               + _FILL * c0 + _C * cnt)

    @pl.when(i == 0)
    def _init():
        out_ref[0, 0] = 0.0

    out_ref[0, 0] += partial


def kernel(x, target):
    tgt3 = target.reshape(_GR, _BR, 1)
    out = pl.pallas_call(
        _body,
        grid=(_GR,),
        in_specs=[
            pl.BlockSpec((1, _BR, 1), lambda i: (i, 0, 0)),
            pl.BlockSpec((_BR, _SIZE), lambda i: (i, 0)),
        ],
        out_specs=pl.BlockSpec(
            (1, 1), lambda i: (0, 0), memory_space=pltpu.SMEM
        ),
        out_shape=jax.ShapeDtypeStruct((1, 1), jnp.float32),
    )(tgt3, x)
    return out[0, 0]
